# baseline (speedup 1.0000x reference)
import functools

import jax
import jax.numpy as jnp
from jax import lax
from jax.experimental import pallas as pl
from jax.experimental.pallas import tpu as pltpu
from jax.experimental.pallas import tpu_sc as plsc

N = 10000
E = 320000
D = 128
ED = 16
EPS = 1e-8

NC = 2
NS = 16
NW = NC * NS
EPW = E // NW
CH = 80
NCH = EPW // CH
NACC = 10240
NPC = 2560
ACCR = 2624
RPT = ACCR // NS
WB = NPC // NS


def _silu(x):
    return x * jax.nn.sigmoid(x)


def _pre_body(h_ref, w1a_ref, w1b_ref, b1_ref, p_ref, q_ref):
    h = h_ref[...]
    p_ref[...] = (jnp.dot(h, w1a_ref[...], preferred_element_type=jnp.float32)
                  + b1_ref[...])
    q_ref[...] = jnp.dot(h, w1b_ref[...], preferred_element_type=jnp.float32)


def _precompute(h, w1a, w1b, b1):
    bn = 1000
    grid = N // bn
    return pl.pallas_call(
        _pre_body,
        grid=(grid,),
        in_specs=[
            pl.BlockSpec((bn, D), lambda i: (i, 0)),
            pl.BlockSpec((D, D), lambda i: (0, 0)),
            pl.BlockSpec((D, D), lambda i: (0, 0)),
            pl.BlockSpec((1, D), lambda i: (0, 0)),
        ],
        out_specs=[
            pl.BlockSpec((bn, D), lambda i: (i, 0)),
            pl.BlockSpec((bn, D), lambda i: (i, 0)),
        ],
        out_shape=[
            jax.ShapeDtypeStruct((N, D), jnp.float32),
            jax.ShapeDtypeStruct((N, D), jnp.float32),
        ],
    )(h, w1a, w1b, b1)


def _gather_body(row_hbm, col_hbm, p_hbm, q_hbm, pos4_hbm, z_hbm, diff_hbm,
                 idxr0, idxc0, idxr1, idxc1, pbuf0, qbuf0, pbuf1, qbuf1,
                 dbuf, posv,
                 semr0, semc0, semr1, semc1, semp0, semq0, semp1, semq1):
    c = lax.axis_index("c")
    s = lax.axis_index("s")
    wid = s * NC + c
    base = wid * EPW
    idxr = (idxr0, idxr1)
    idxc = (idxc0, idxc1)
    pbuf = (pbuf0, pbuf1)
    qbuf = (qbuf0, qbuf1)
    semr = (semr0, semr1)
    semc = (semc0, semc1)
    semp = (semp0, semp1)
    semq = (semq0, semq1)

    pltpu.sync_copy(pos4_hbm, posv)

    @pl.loop(0, CH)
    def _zd(r):
        dbuf[r, :] = jnp.zeros((16,), jnp.float32)

    def issue_idx(ch, b):
        eb = base + ch * CH
        pltpu.async_copy(row_hbm.at[pl.ds(eb, CH)], idxr[b], semr[b])
        pltpu.async_copy(col_hbm.at[pl.ds(eb, CH)], idxc[b], semc[b])

    def wait_idx(b):
        pltpu.make_async_copy(row_hbm.at[pl.ds(0, CH)], idxr[b], semr[b]).wait()
        pltpu.make_async_copy(col_hbm.at[pl.ds(0, CH)], idxc[b], semc[b]).wait()

    def issue_gather(b):
        pltpu.async_copy(p_hbm.at[idxr[b]], pbuf[b], semp[b])
        pltpu.async_copy(q_hbm.at[idxc[b]], qbuf[b], semq[b])

    def wait_gather(b):
        pltpu.make_async_copy(p_hbm.at[pl.ds(0, CH)], pbuf[b], semp[b]).wait()
        pltpu.make_async_copy(q_hbm.at[pl.ds(0, CH)], qbuf[b], semq[b]).wait()

    issue_idx(0, 0)
    wait_idx(0)
    issue_gather(0)
    issue_idx(1, 1)

    lanes = lax.iota(jnp.int32, 16)

    def work(ch, b):
        o = 1 - b
        @pl.when(ch + 1 < NCH)
        def _():
            wait_idx(o)
            issue_gather(o)

        for g in range(CH // 16):
            ir = idxr[b][pl.ds(g * 16, 16)] * 4
            ic = idxc[b][pl.ds(g * 16, 16)] * 4
            rows = lanes + g * 16
            for j in range(3):
                jv = jnp.full((16,), j, jnp.int32)
                dr = (plsc.load_gather(posv, [ir + j])
                      - plsc.load_gather(posv, [ic + j]))
                plsc.store_scatter(dbuf, [rows, jv], dr)

        wait_gather(b)

        @pl.loop(0, CH)
        def _row(r):
            for j in range(D // 16):
                sl = pl.ds(j * 16, 16)
                pbuf[b][r, sl] = pbuf[b][r, sl] + qbuf[b][r, sl]

        eb = base + ch * CH
        pltpu.sync_copy(pbuf[b], z_hbm.at[pl.ds(eb, CH)])
        pltpu.sync_copy(dbuf, diff_hbm.at[pl.ds(eb, CH)])

        @pl.when(ch + 2 < NCH)
        def _():
            issue_idx(ch + 2, b)

    @pl.loop(0, NCH, step=2)
    def _pair(ch0):
        work(ch0, 0)
        pl.when(ch0 + 1 < NCH)(lambda: work(ch0 + 1, 1))


def _gather(row, col, p, q, pos4):
    mesh = plsc.VectorSubcoreMesh(
        core_axis_name="c", subcore_axis_name="s",
        num_cores=NC, num_subcores=NS)
    f = pl.kernel(
        _gather_body,
        out_type=[
            jax.ShapeDtypeStruct((E, D), jnp.float32),
            jax.ShapeDtypeStruct((E, 16), jnp.float32),
        ],
        mesh=mesh,
        scratch_types=(
            [pltpu.VMEM((CH,), jnp.int32)] * 4
            + [pltpu.VMEM((CH, D), jnp.float32)] * 4
            + [pltpu.VMEM((CH, 16), jnp.float32),
               pltpu.VMEM((4 * N,), jnp.float32)]
            + [pltpu.SemaphoreType.DMA] * 8
        ),
        compiler_params=pltpu.CompilerParams(needs_layout_passes=False),
    )
    return f(row, col, p, q, pos4)


def _bdot(a, b):
    return jnp.dot(a.astype(jnp.bfloat16), b.astype(jnp.bfloat16),
                   preferred_element_type=jnp.float32)


def _edge_body(z_ref, ea_ref, diff_ref, w1c_ref, w1r_ref,
               w2_ref, b2_ref, wc1_ref, bc1_ref, wc2_ref,
               e_ref, trans_ref):
    diff = diff_ref[...]
    r2 = jnp.sum(diff[:, :3] * diff[:, :3], axis=1, keepdims=True)
    radial = jnp.sqrt(r2) + EPS
    t = (z_ref[...]
         + jnp.dot(ea_ref[...], w1c_ref[...],
                   preferred_element_type=jnp.float32)
         + radial * w1r_ref[...])
    u = _silu(t)
    e = _silu(_bdot(u, w2_ref[...]) + b2_ref[...])
    g = _silu(_bdot(e, wc1_ref[...]) + bc1_ref[...])
    cu = jnp.dot(g, wc2_ref[...], preferred_element_type=jnp.float32)
    cu = jnp.clip(cu, -1.0, 1.0)
    e_ref[...] = e
    trans_ref[...] = cu * diff


def _edge_mlp(z, ea, diff, w1c, w1r, w2, b2, wc1, bc1, wc2):
    be = 512
    grid = E // be
    full = lambda w: pl.BlockSpec(w.shape, lambda i: tuple(0 for _ in w.shape))
    return pl.pallas_call(
        _edge_body,
        grid=(grid,),
        in_specs=[
            pl.BlockSpec((be, D), lambda i: (i, 0)),
            pl.BlockSpec((be, ED), lambda i: (i, 0)),
            pl.BlockSpec((be, 16), lambda i: (i, 0)),
            full(w1c), full(w1r), full(w2), full(b2),
            full(wc1), full(bc1), full(wc2),
        ],
        out_specs=[
            pl.BlockSpec((be, D), lambda i: (i, 0)),
            pl.BlockSpec((be, 16), lambda i: (i, 0)),
        ],
        out_shape=[
            jax.ShapeDtypeStruct((E, D), jnp.float32),
            jax.ShapeDtypeStruct((E, 16), jnp.float32),
        ],
    )(z, ea, diff, w1c, w1r, w2, b2, wc1, bc1, wc2)


def _scatter_body(nbase, row_hbm, e_hbm, trans_hbm, nodep_hbm, coordp_hbm,
                  idx0, idx1, ebuf0, ebuf1, tbuf0, tbuf1, zbuf, accv, acc_n,
                  semi0, semi1, seme0, seme1, semt0, semt1):
    c = lax.axis_index("c")
    s = lax.axis_index("s")
    lo = nbase + c * NPC
    lanes = lax.iota(jnp.int32, 16)
    idx = (idx0, idx1)
    ebuf = (ebuf0, ebuf1)
    tbuf = (tbuf0, tbuf1)
    semi = (semi0, semi1)
    seme = (seme0, seme1)
    semt = (semt0, semt1)
    NCH2 = E // NS // CH

    @pl.loop(0, RPT)
    def _zrow(r):
        for j in range(D // 16):
            zbuf[r, pl.ds(j * 16, 16)] = jnp.zeros((16,), jnp.float32)

    pltpu.sync_copy(zbuf, acc_n.at[pl.ds(s * RPT, RPT)])

    @pl.loop(0, NPC + 8)
    def _zcrow(r):
        accv[pl.ds(r * 16, 16)] = jnp.zeros((16,), jnp.float32)

    plsc.subcore_barrier()

    def issue(ch, b):
        eb = s * (E // NS) + ch * CH
        pltpu.async_copy(row_hbm.at[pl.ds(eb, CH)], idx[b], semi[b])
        pltpu.async_copy(e_hbm.at[pl.ds(eb, CH)], ebuf[b], seme[b])
        pltpu.async_copy(trans_hbm.at[pl.ds(eb, CH)], tbuf[b], semt[b])

    def wait(b):
        pltpu.make_async_copy(row_hbm.at[pl.ds(0, CH)], idx[b], semi[b]).wait()
        pltpu.make_async_copy(e_hbm.at[pl.ds(0, CH)], ebuf[b], seme[b]).wait()
        pltpu.make_async_copy(trans_hbm.at[pl.ds(0, CH)], tbuf[b],
                              semt[b]).wait()

    issue(0, 0)
    issue(1, 1)

    @pl.loop(0, NCH2, step=2)
    def _pair(ch0):
        for b in range(2):
            ch = ch0 + b
            wait(b)
            for g in range(CH // 16):
                sl = pl.ds(g * 16, 16)
                v = idx[b][sl] - lo
                valid = (v >= 0) & (v < NPC)
                v = jnp.where(valid, v, NPC)
                idx[b][sl] = v
                rows = lanes + g * 16
                vf = v * 16
                for j in range(3):
                    jv = jnp.full((16,), j, jnp.int32)
                    tj = plsc.load_gather(tbuf[b], [rows, jv])
                    plsc.addupdate_scatter(accv, [vf + j], tj)
            pltpu.sync_copy(ebuf[b], acc_n.at[idx[b]], add=True)

            @pl.when(ch + 2 < NCH2)
            def _():
                issue(ch + 2, b)

    plsc.subcore_barrier()

    nb = s * WB
    ob = c * NPC + nb
    pltpu.sync_copy(acc_n.at[pl.ds(nb, WB)], zbuf.at[pl.ds(0, WB)])
    pltpu.sync_copy(zbuf.at[pl.ds(0, WB)], nodep_hbm.at[pl.ds(ob, WB)])
    pltpu.sync_copy(accv.at[pl.ds(0, NPC * 16)], coordp_hbm.at[c, s])


def _scatter(row, e, trans, nbase):
    mesh = plsc.VectorSubcoreMesh(
        core_axis_name="c", subcore_axis_name="s",
        num_cores=NC, num_subcores=NS)
    f = pl.kernel(
        functools.partial(_scatter_body, nbase),
        out_type=[
            jax.ShapeDtypeStruct((2 * NPC, D), jnp.float32),
            jax.ShapeDtypeStruct((NC, NS, NPC * 16), jnp.float32),
        ],
        mesh=mesh,
        scratch_types=(
            [pltpu.VMEM((CH,), jnp.int32)] * 2
            + [pltpu.VMEM((CH, D), jnp.float32)] * 2
            + [pltpu.VMEM((CH, 16), jnp.float32)] * 2
            + [pltpu.VMEM((RPT, D), jnp.float32),
               pltpu.VMEM(((NPC + 8) * 16,), jnp.float32),
               pltpu.VMEM_SHARED((ACCR, D), jnp.float32)]
            + [pltpu.SemaphoreType.DMA] * 6
        ),
        compiler_params=pltpu.CompilerParams(needs_layout_passes=False),
    )
    return f(row, e, trans)


def _node_body(h_ref, np_ref, cp_ref, pos_ref, wn1a_ref, wn1b_ref, bn1_ref,
               wn2_ref, bn2_ref, h_out, pos_out):
    h = h_ref[...]
    agg = np_ref[...]
    u = _silu(_bdot(h, wn1a_ref[...]) + _bdot(agg, wn1b_ref[...])
              + bn1_ref[...])
    h_out[...] = _bdot(u, wn2_ref[...]) + bn2_ref[...] + h
    pc = jnp.sum(cp_ref[...], axis=0)
    pos_out[...] = pos_ref[...] + pc[:, :3]


def _node_mlp(h, nodep, coordp, pos, wn1a, wn1b, bn1, wn2, bn2):
    bn = 1000
    grid = N // bn
    full = lambda w: pl.BlockSpec(w.shape, lambda i: tuple(0 for _ in w.shape))
    return pl.pallas_call(
        _node_body,
        grid=(grid,),
        in_specs=[
            pl.BlockSpec((bn, D), lambda i: (i, 0)),
            pl.BlockSpec((bn, D), lambda i: (i, 0)),
            pl.BlockSpec((NS, bn, 16), lambda i: (0, i, 0)),
            pl.BlockSpec((bn, 3), lambda i: (i, 0)),
            full(wn1a), full(wn1b), full(bn1), full(wn2), full(bn2),
        ],
        out_specs=[
            pl.BlockSpec((bn, D), lambda i: (i, 0)),
            pl.BlockSpec((bn, 3), lambda i: (i, 0)),
        ],
        out_shape=[
            jax.ShapeDtypeStruct((N, D), jnp.float32),
            jax.ShapeDtypeStruct((N, 3), jnp.float32),
        ],
    )(h, nodep, coordp, pos, wn1a, wn1b, bn1, wn2, bn2)


def kernel(h, edge_index, edge_attr, pos, W1, b1, W2, b2, Wc1, bc1, Wc2,
           Wn1, bn1, Wn2, bn2):
    row = edge_index[0]
    col = edge_index[1]
    w1a, w1b, w1c, w1r = (W1[:D], W1[D:2 * D], W1[2 * D:2 * D + ED],
                          W1[2 * D + ED:])
    pos4 = jnp.pad(pos, ((0, 0), (0, 1))).reshape(-1)
    p, q = _precompute(h, w1a, w1b, b1.reshape(1, D))
    z, diff = _gather(row, col, p, q, pos4)
    e, trans = _edge_mlp(z, edge_attr, diff, w1c, w1r.reshape(1, D),
                         W2, b2.reshape(1, D),
                         Wc1, bc1.reshape(1, D), Wc2)
    np0, cp0 = _scatter(row, e, trans, 0)
    np1, cp1 = _scatter(row, e, trans, 2 * NPC)
    nodep = jnp.concatenate([np0, np1], axis=0)
    coordp = (jnp.stack([cp0, cp1]).reshape(2, NC, NS, NPC, 16)
              .transpose(2, 0, 1, 3, 4).reshape(NS, NACC, 16))
    h_new, pos_new = _node_mlp(h, nodep, coordp, pos,
                               Wn1[:D], Wn1[D:], bn1.reshape(1, D),
                               Wn2, bn2.reshape(1, D))
    return (h_new, pos_new)


# speedup vs baseline: 1.2036x; 1.2036x over previous
import functools

import jax
import jax.numpy as jnp
from jax import lax
from jax.experimental import pallas as pl
from jax.experimental.pallas import tpu as pltpu
from jax.experimental.pallas import tpu_sc as plsc

N = 10000
E = 320000
D = 128
ED = 16
EPS = 1e-8

NC = 2
NS = 16
NW = NC * NS
EPW = E // NW
CH = 80
NCH = EPW // CH
NACC = 10240
NPC = 2560
ACCR = 2624
RPT = ACCR // NS
WB = NPC // NS


def _silu(x):
    return x * jax.nn.sigmoid(x)


def _pre_body(h_ref, w1a_ref, w1b_ref, b1_ref, p_ref, q_ref):
    h = h_ref[...]
    p_ref[...] = (jnp.dot(h, w1a_ref[...], preferred_element_type=jnp.float32)
                  + b1_ref[...])
    q_ref[...] = jnp.dot(h, w1b_ref[...], preferred_element_type=jnp.float32)


def _precompute(h, w1a, w1b, b1):
    bn = 1000
    grid = N // bn
    return pl.pallas_call(
        _pre_body,
        grid=(grid,),
        in_specs=[
            pl.BlockSpec((bn, D), lambda i: (i, 0)),
            pl.BlockSpec((D, D), lambda i: (0, 0)),
            pl.BlockSpec((D, D), lambda i: (0, 0)),
            pl.BlockSpec((1, D), lambda i: (0, 0)),
        ],
        out_specs=[
            pl.BlockSpec((bn, D), lambda i: (i, 0)),
            pl.BlockSpec((bn, D), lambda i: (i, 0)),
        ],
        out_shape=[
            jax.ShapeDtypeStruct((N, D), jnp.float32),
            jax.ShapeDtypeStruct((N, D), jnp.float32),
        ],
    )(h, w1a, w1b, b1)


def _gather_body(epw, nch, row_hbm, col_hbm, p_hbm, q_hbm, pos4_hbm, z_hbm,
                 diff_hbm,
                 idxr0, idxc0, idxr1, idxc1, pbuf0, qbuf0, pbuf1, qbuf1,
                 dbuf, posv,
                 semr0, semc0, semr1, semc1, semp0, semq0, semp1, semq1):
    c = lax.axis_index("c")
    s = lax.axis_index("s")
    wid = s * NC + c
    base = wid * epw
    idxr = (idxr0, idxr1)
    idxc = (idxc0, idxc1)
    pbuf = (pbuf0, pbuf1)
    qbuf = (qbuf0, qbuf1)
    semr = (semr0, semr1)
    semc = (semc0, semc1)
    semp = (semp0, semp1)
    semq = (semq0, semq1)

    pltpu.sync_copy(pos4_hbm, posv)

    @pl.loop(0, CH)
    def _zd(r):
        dbuf[r, :] = jnp.zeros((16,), jnp.float32)

    def issue_idx(ch, b):
        eb = base + ch * CH
        pltpu.async_copy(row_hbm.at[pl.ds(eb, CH)], idxr[b], semr[b])
        pltpu.async_copy(col_hbm.at[pl.ds(eb, CH)], idxc[b], semc[b])

    def wait_idx(b):
        pltpu.make_async_copy(row_hbm.at[pl.ds(0, CH)], idxr[b], semr[b]).wait()
        pltpu.make_async_copy(col_hbm.at[pl.ds(0, CH)], idxc[b], semc[b]).wait()

    def issue_gather(b):
        pltpu.async_copy(p_hbm.at[idxr[b]], pbuf[b], semp[b])
        pltpu.async_copy(q_hbm.at[idxc[b]], qbuf[b], semq[b])

    def wait_gather(b):
        pltpu.make_async_copy(p_hbm.at[pl.ds(0, CH)], pbuf[b], semp[b]).wait()
        pltpu.make_async_copy(q_hbm.at[pl.ds(0, CH)], qbuf[b], semq[b]).wait()

    issue_idx(0, 0)
    wait_idx(0)
    issue_gather(0)
    issue_idx(1, 1)

    lanes = lax.iota(jnp.int32, 16)

    def work(ch, b):
        o = 1 - b
        @pl.when(ch + 1 < nch)
        def _():
            wait_idx(o)
            issue_gather(o)

        for g in range(CH // 16):
            ir = idxr[b][pl.ds(g * 16, 16)] * 4
            ic = idxc[b][pl.ds(g * 16, 16)] * 4
            rows = lanes + g * 16
            for j in range(3):
                jv = jnp.full((16,), j, jnp.int32)
                dr = (plsc.load_gather(posv, [ir + j])
                      - plsc.load_gather(posv, [ic + j]))
                plsc.store_scatter(dbuf, [rows, jv], dr)

        wait_gather(b)

        @pl.loop(0, CH)
        def _row(r):
            for j in range(D // 16):
                sl = pl.ds(j * 16, 16)
                pbuf[b][r, sl] = pbuf[b][r, sl] + qbuf[b][r, sl]

        eb = base + ch * CH
        pltpu.sync_copy(pbuf[b], z_hbm.at[pl.ds(eb, CH)])
        pltpu.sync_copy(dbuf, diff_hbm.at[pl.ds(eb, CH)])

        @pl.when(ch + 2 < nch)
        def _():
            issue_idx(ch + 2, b)

    @pl.loop(0, nch, step=2)
    def _pair(ch0):
        work(ch0, 0)
        pl.when(ch0 + 1 < nch)(lambda: work(ch0 + 1, 1))


def _gather(row, col, p, q, pos4):
    eh = row.shape[0]
    epw = eh // NW
    nch = epw // CH
    mesh = plsc.VectorSubcoreMesh(
        core_axis_name="c", subcore_axis_name="s",
        num_cores=NC, num_subcores=NS)
    f = pl.kernel(
        functools.partial(_gather_body, epw, nch),
        out_type=[
            jax.ShapeDtypeStruct((eh, D), jnp.float32),
            jax.ShapeDtypeStruct((eh, 16), jnp.float32),
        ],
        mesh=mesh,
        scratch_types=(
            [pltpu.VMEM((CH,), jnp.int32)] * 4
            + [pltpu.VMEM((CH, D), jnp.float32)] * 4
            + [pltpu.VMEM((CH, 16), jnp.float32),
               pltpu.VMEM((4 * N,), jnp.float32)]
            + [pltpu.SemaphoreType.DMA] * 8
        ),
        compiler_params=pltpu.CompilerParams(needs_layout_passes=False),
    )
    return f(row, col, p, q, pos4)


def _bdot(a, b):
    return jnp.dot(a.astype(jnp.bfloat16), b.astype(jnp.bfloat16),
                   preferred_element_type=jnp.float32)


def _edge_body(z_ref, ea_ref, diff_ref, w1c_ref, w1r_ref,
               w2_ref, b2_ref, wc1_ref, bc1_ref, wc2_ref,
               e_ref, trans_ref):
    diff = diff_ref[...]
    r2 = jnp.sum(diff[:, :3] * diff[:, :3], axis=1, keepdims=True)
    radial = jnp.sqrt(r2) + EPS
    t = (z_ref[...]
         + jnp.dot(ea_ref[...], w1c_ref[...],
                   preferred_element_type=jnp.float32)
         + radial * w1r_ref[...])
    u = _silu(t)
    e = _silu(_bdot(u, w2_ref[...]) + b2_ref[...])
    g = _silu(_bdot(e, wc1_ref[...]) + bc1_ref[...])
    cu = jnp.dot(g, wc2_ref[...], preferred_element_type=jnp.float32)
    cu = jnp.clip(cu, -1.0, 1.0)
    e_ref[...] = e
    trans_ref[...] = cu * diff


def _edge_mlp(z, ea, diff, w1c, w1r, w2, b2, wc1, bc1, wc2):
    be = 512
    grid = z.shape[0] // be
    full = lambda w: pl.BlockSpec(w.shape, lambda i: tuple(0 for _ in w.shape))
    return pl.pallas_call(
        _edge_body,
        grid=(grid,),
        in_specs=[
            pl.BlockSpec((be, D), lambda i: (i, 0)),
            pl.BlockSpec((be, ED), lambda i: (i, 0)),
            pl.BlockSpec((be, 16), lambda i: (i, 0)),
            full(w1c), full(w1r), full(w2), full(b2),
            full(wc1), full(bc1), full(wc2),
        ],
        out_specs=[
            pl.BlockSpec((be, D), lambda i: (i, 0)),
            pl.BlockSpec((be, 16), lambda i: (i, 0)),
        ],
        out_shape=[
            jax.ShapeDtypeStruct((z.shape[0], D), jnp.float32),
            jax.ShapeDtypeStruct((z.shape[0], 16), jnp.float32),
        ],
    )(z, ea, diff, w1c, w1r, w2, b2, wc1, bc1, wc2)


def _scatter_body(nbase, epw2, row_hbm, e_hbm, trans_hbm, nodep_hbm,
                  coordp_hbm,
                  idx0, idx1, ebuf0, ebuf1, tbuf0, tbuf1, zbuf, accv, acc_n,
                  semi0, semi1, seme0, seme1, semt0, semt1):
    c = lax.axis_index("c")
    s = lax.axis_index("s")
    lo = nbase + c * NPC
    lanes = lax.iota(jnp.int32, 16)
    idx = (idx0, idx1)
    ebuf = (ebuf0, ebuf1)
    tbuf = (tbuf0, tbuf1)
    semi = (semi0, semi1)
    seme = (seme0, seme1)
    semt = (semt0, semt1)
    NCH2 = epw2 // CH

    @pl.loop(0, RPT)
    def _zrow(r):
        for j in range(D // 16):
            zbuf[r, pl.ds(j * 16, 16)] = jnp.zeros((16,), jnp.float32)

    pltpu.sync_copy(zbuf, acc_n.at[pl.ds(s * RPT, RPT)])

    @pl.loop(0, NPC + 8)
    def _zcrow(r):
        accv[pl.ds(r * 16, 16)] = jnp.zeros((16,), jnp.float32)

    plsc.subcore_barrier()

    def issue(ch, b):
        eb = s * epw2 + ch * CH
        pltpu.async_copy(row_hbm.at[pl.ds(eb, CH)], idx[b], semi[b])
        pltpu.async_copy(e_hbm.at[pl.ds(eb, CH)], ebuf[b], seme[b])
        pltpu.async_copy(trans_hbm.at[pl.ds(eb, CH)], tbuf[b], semt[b])

    def wait(b):
        pltpu.make_async_copy(row_hbm.at[pl.ds(0, CH)], idx[b], semi[b]).wait()
        pltpu.make_async_copy(e_hbm.at[pl.ds(0, CH)], ebuf[b], seme[b]).wait()
        pltpu.make_async_copy(trans_hbm.at[pl.ds(0, CH)], tbuf[b],
                              semt[b]).wait()

    issue(0, 0)
    issue(1, 1)

    @pl.loop(0, NCH2, step=2)
    def _pair(ch0):
        for b in range(2):
            ch = ch0 + b
            wait(b)
            for g in range(CH // 16):
                sl = pl.ds(g * 16, 16)
                v = idx[b][sl] - lo
                valid = (v >= 0) & (v < NPC)
                v = jnp.where(valid, v, NPC)
                idx[b][sl] = v
                rows = lanes + g * 16
                vf = v * 16
                for j in range(3):
                    jv = jnp.full((16,), j, jnp.int32)
                    tj = plsc.load_gather(tbuf[b], [rows, jv])
                    plsc.addupdate_scatter(accv, [vf + j], tj)
            pltpu.sync_copy(ebuf[b], acc_n.at[idx[b]], add=True)

            @pl.when(ch + 2 < NCH2)
            def _():
                issue(ch + 2, b)

    plsc.subcore_barrier()

    nb = s * WB
    ob = c * NPC + nb
    pltpu.sync_copy(acc_n.at[pl.ds(nb, WB)], zbuf.at[pl.ds(0, WB)])
    pltpu.sync_copy(zbuf.at[pl.ds(0, WB)], nodep_hbm.at[pl.ds(ob, WB)])
    pltpu.sync_copy(accv.at[pl.ds(0, NPC * 16)], coordp_hbm.at[c, s])


def _scatter(row, e, trans, nbase):
    epw2 = row.shape[0] // NS
    mesh = plsc.VectorSubcoreMesh(
        core_axis_name="c", subcore_axis_name="s",
        num_cores=NC, num_subcores=NS)
    f = pl.kernel(
        functools.partial(_scatter_body, nbase, epw2),
        out_type=[
            jax.ShapeDtypeStruct((2 * NPC, D), jnp.float32),
            jax.ShapeDtypeStruct((NC, NS, NPC * 16), jnp.float32),
        ],
        mesh=mesh,
        scratch_types=(
            [pltpu.VMEM((CH,), jnp.int32)] * 2
            + [pltpu.VMEM((CH, D), jnp.float32)] * 2
            + [pltpu.VMEM((CH, 16), jnp.float32)] * 2
            + [pltpu.VMEM((RPT, D), jnp.float32),
               pltpu.VMEM(((NPC + 8) * 16,), jnp.float32),
               pltpu.VMEM_SHARED((ACCR, D), jnp.float32)]
            + [pltpu.SemaphoreType.DMA] * 6
        ),
        compiler_params=pltpu.CompilerParams(needs_layout_passes=False),
    )
    return f(row, e, trans)


def _node_body(h_ref, npa_ref, npb_ref, cpa_ref, cpb_ref, pos_ref, wn1a_ref,
               wn1b_ref, bn1_ref, wn2_ref, bn2_ref, h_out, pos_out):
    h = h_ref[...]
    agg = npa_ref[...] + npb_ref[...]
    u = _silu(_bdot(h, wn1a_ref[...]) + _bdot(agg, wn1b_ref[...])
              + bn1_ref[...])
    h_out[...] = _bdot(u, wn2_ref[...]) + bn2_ref[...] + h
    pc = jnp.sum(cpa_ref[...], axis=0) + jnp.sum(cpb_ref[...], axis=0)
    pos_out[...] = pos_ref[...] + pc[:, :3]


def _node_mlp(h, npa, npb, cpa, cpb, pos, wn1a, wn1b, bn1, wn2, bn2):
    bn = 1000
    grid = N // bn
    full = lambda w: pl.BlockSpec(w.shape, lambda i: tuple(0 for _ in w.shape))
    return pl.pallas_call(
        _node_body,
        grid=(grid,),
        in_specs=[
            pl.BlockSpec((bn, D), lambda i: (i, 0)),
            pl.BlockSpec((bn, D), lambda i: (i, 0)),
            pl.BlockSpec((bn, D), lambda i: (i, 0)),
            pl.BlockSpec((NS, bn, 16), lambda i: (0, i, 0)),
            pl.BlockSpec((NS, bn, 16), lambda i: (0, i, 0)),
            pl.BlockSpec((bn, 3), lambda i: (i, 0)),
            full(wn1a), full(wn1b), full(bn1), full(wn2), full(bn2),
        ],
        out_specs=[
            pl.BlockSpec((bn, D), lambda i: (i, 0)),
            pl.BlockSpec((bn, 3), lambda i: (i, 0)),
        ],
        out_shape=[
            jax.ShapeDtypeStruct((N, D), jnp.float32),
            jax.ShapeDtypeStruct((N, 3), jnp.float32),
        ],
    )(h, npa, npb, cpa, cpb, pos, wn1a, wn1b, bn1, wn2, bn2)


def kernel(h, edge_index, edge_attr, pos, W1, b1, W2, b2, Wc1, bc1, Wc2,
           Wn1, bn1, Wn2, bn2):
    row = edge_index[0]
    col = edge_index[1]
    w1a, w1b, w1c, w1r = (W1[:D], W1[D:2 * D], W1[2 * D:2 * D + ED],
                          W1[2 * D + ED:])
    pos4 = jnp.pad(pos, ((0, 0), (0, 1))).reshape(-1)
    p, q = _precompute(h, w1a, w1b, b1.reshape(1, D))

    EH0 = 163840
    halves = []
    for lo_e, hi_e in ((0, EH0), (EH0, E)):
        rowh, colh = row[lo_e:hi_e], col[lo_e:hi_e]
        z, diff = _gather(rowh, colh, p, q, pos4)
        e, trans = _edge_mlp(z, edge_attr[lo_e:hi_e], diff, w1c,
                             w1r.reshape(1, D), W2, b2.reshape(1, D),
                             Wc1, bc1.reshape(1, D), Wc2)
        npl = _scatter(rowh, e, trans, 0)
        nph = _scatter(rowh, e, trans, 2 * NPC)
        halves.append((npl, nph))

    def assemble(npl, nph):
        nodep = jnp.concatenate([npl[0], nph[0]], axis=0)
        coordp = (jnp.stack([npl[1], nph[1]]).reshape(2, NC, NS, NPC, 16)
                  .transpose(2, 0, 1, 3, 4).reshape(NS, NACC, 16))
        return nodep, coordp

    npa, cpa = assemble(*halves[0])
    npb, cpb = assemble(*halves[1])
    h_new, pos_new = _node_mlp(h, npa, npb, cpa, cpb, pos,
                               Wn1[:D], Wn1[D:], bn1.reshape(1, D),
                               Wn2, bn2.reshape(1, D))
    return (h_new, pos_new)
